# baseline (speedup 1.0000x reference)
import functools

import jax
import jax.numpy as jnp
from jax import lax
from jax.experimental import pallas as pl
from jax.experimental.pallas import tpu as pltpu
from jax.experimental.pallas import tpu_sc as plsc

_LANES = 16
_NUM_CORES = 2
_NUM_SUBCORES = 16
_VOCAB_PAD = 1024
_NCHUNK = 4


def _vtable_tc_kernel(emb_ref, fcw_ref, out_ref):
    v = lax.dot_general(
        emb_ref[...], fcw_ref[...],
        (((1,), (1,)), ((), ())),
        preferred_element_type=jnp.float32,
    )
    out_ref[...] = jnp.zeros_like(out_ref)
    out_ref[0:emb_ref.shape[0], :] = v


def kernel(input_x, char_emb, weight_char_emb, fc1_w, fc1_b):
    B, M = input_x.shape
    V, E = char_emb.shape
    NW = _NUM_CORES * _NUM_SUBCORES
    BPW = B // NW
    CW = BPW // _NCHUNK
    GPC = CW // _LANES
    MCH = (M + _LANES - 1) // _LANES

    v_tab = pl.pallas_call(
        _vtable_tc_kernel,
        out_shape=jax.ShapeDtypeStruct((_VOCAB_PAD, 1), jnp.float32),
    )(char_emb, fc1_w)
    v_tab = v_tab.reshape(_VOCAB_PAD)

    xt = input_x.T

    mesh = plsc.VectorSubcoreMesh(core_axis_name="c", subcore_axis_name="s")

    @functools.partial(
        pl.kernel,
        out_type=jax.ShapeDtypeStruct((B,), jnp.float32),
        mesh=mesh,
        compiler_params=pltpu.CompilerParams(needs_layout_passes=False),
        scratch_types=[
            pltpu.VMEM((2, M, CW), jnp.int32),
            pltpu.VMEM((_VOCAB_PAD,), jnp.float32),
            pltpu.VMEM((MCH * _LANES,), jnp.float32),
            pltpu.VMEM((_LANES,), jnp.float32),
            pltpu.VMEM((BPW,), jnp.float32),
            pltpu.SemaphoreType.DMA,
            pltpu.SemaphoreType.DMA,
        ],
    )
    def sc_score(xt_hbm, v_hbm, w_hbm, b_hbm, out_hbm,
                 x_v, v_v, w_v, b_v, o_v, sem0, sem1):
        wid = lax.axis_index("s") * _NUM_CORES + lax.axis_index("c")
        base = wid * BPW
        sems = (sem0, sem1)

        pltpu.sync_copy(w_hbm, w_v.at[0:M])
        pltpu.sync_copy(b_hbm, b_v.at[0:1])
        pltpu.sync_copy(v_hbm, v_v)

        def start_chunk(c):
            return pltpu.async_copy(
                xt_hbm.at[:, pl.ds(base + c * CW, CW)],
                x_v.at[c % 2], sems[c % 2])

        pending = start_chunk(0)
        bias = b_v[pl.ds(0, _LANES)][0]
        w_chunks = [w_v[pl.ds(k * _LANES, _LANES)] for k in range(MCH)]

        for c in range(_NCHUNK):
            nxt = start_chunk(c + 1) if c + 1 < _NCHUNK else None
            pending.wait()
            xc = x_v.at[c % 2]

            def g_body(g, carry):
                gb = g * _LANES
                acc = jnp.zeros((_LANES,), jnp.float32)
                for m in range(M):
                    idx = xc[m, pl.ds(gb, _LANES)]
                    gv = plsc.load_gather(v_v, [idx])
                    acc = acc + gv * w_chunks[m // _LANES][m % _LANES]
                o_v[pl.ds(c * CW + gb, _LANES)] = acc + bias
                return carry

            lax.fori_loop(0, GPC, g_body, 0)
            pending = nxt

        pltpu.sync_copy(o_v, out_hbm.at[pl.ds(base, BPW)])

    return sc_score(xt, v_tab, weight_char_emb, fc1_b)


# speedup vs baseline: 1.2431x; 1.2431x over previous
import functools

import jax
import jax.numpy as jnp
from jax import lax
from jax.experimental import pallas as pl
from jax.experimental.pallas import tpu as pltpu
from jax.experimental.pallas import tpu_sc as plsc

_LANES = 16
_NUM_CORES = 2
_NUM_SUBCORES = 16
_VOCAB_PAD = 1024
_NCHUNK = 4


def _vtable_tc_kernel(emb_ref, fcw_ref, out_ref):
    v = lax.dot_general(
        emb_ref[...], fcw_ref[...],
        (((1,), (1,)), ((), ())),
        preferred_element_type=jnp.float32,
    )
    out_ref[...] = jnp.zeros_like(out_ref)
    out_ref[0:emb_ref.shape[0], :] = v


def kernel(input_x, char_emb, weight_char_emb, fc1_w, fc1_b):
    B, M = input_x.shape
    V, E = char_emb.shape
    NW = _NUM_CORES * _NUM_SUBCORES
    BPW = B // NW
    CW = BPW // _NCHUNK
    GPC = CW // _LANES
    MCH = (M + _LANES - 1) // _LANES

    v_tab = pl.pallas_call(
        _vtable_tc_kernel,
        out_shape=jax.ShapeDtypeStruct((_VOCAB_PAD, 1), jnp.float32),
    )(char_emb, fc1_w)
    v_tab = v_tab.reshape(_VOCAB_PAD)

    xt = input_x.T

    mesh = plsc.VectorSubcoreMesh(core_axis_name="c", subcore_axis_name="s")

    @functools.partial(
        pl.kernel,
        out_type=jax.ShapeDtypeStruct((B,), jnp.float32),
        mesh=mesh,
        compiler_params=pltpu.CompilerParams(needs_layout_passes=False),
        scratch_types=[
            pltpu.VMEM((2, M, CW), jnp.int32),
            pltpu.VMEM((_VOCAB_PAD,), jnp.float32),
            pltpu.VMEM((MCH * _LANES,), jnp.float32),
            pltpu.VMEM((_LANES,), jnp.float32),
            pltpu.VMEM((BPW,), jnp.float32),
            pltpu.SemaphoreType.DMA,
            pltpu.SemaphoreType.DMA,
        ],
    )
    def sc_score(xt_hbm, v_hbm, w_hbm, b_hbm, out_hbm,
                 x_v, v_v, w_v, b_v, o_v, sem0, sem1):
        wid = lax.axis_index("s") * _NUM_CORES + lax.axis_index("c")
        base = wid * BPW
        sems = (sem0, sem1)

        pltpu.sync_copy(w_hbm, w_v.at[0:M])
        pltpu.sync_copy(b_hbm, b_v.at[0:1])
        pltpu.sync_copy(v_hbm, v_v)

        def start_chunk(c):
            return pltpu.async_copy(
                xt_hbm.at[:, pl.ds(base + c * CW, CW)],
                x_v.at[c % 2], sems[c % 2])

        pending = start_chunk(0)
        bias = b_v[pl.ds(0, _LANES)][0]
        MFULL = M // _LANES
        MTAIL = M % _LANES
        w_tail = w_v[pl.ds(MFULL * _LANES, _LANES)]

        for c in range(_NCHUNK):
            nxt = start_chunk(c + 1) if c + 1 < _NCHUNK else None
            pending.wait()
            xc = x_v.at[c % 2]

            def g_body(g, carry):
                gb = g * _LANES

                def mc_body(mc, acc):
                    wc = w_v[pl.ds(mc * _LANES, _LANES)]
                    mb = mc * _LANES
                    for i in range(_LANES):
                        idx = xc[mb + i, pl.ds(gb, _LANES)]
                        gv = plsc.load_gather(v_v, [idx])
                        acc = acc + gv * wc[i]
                    return acc

                acc = lax.fori_loop(
                    0, MFULL, mc_body, jnp.zeros((_LANES,), jnp.float32))
                for i in range(MTAIL):
                    idx = xc[MFULL * _LANES + i, pl.ds(gb, _LANES)]
                    gv = plsc.load_gather(v_v, [idx])
                    acc = acc + gv * w_tail[i]
                o_v[pl.ds(c * CW + gb, _LANES)] = acc + bias
                return carry

            lax.fori_loop(0, GPC, g_body, 0)
            pending = nxt

        pltpu.sync_copy(o_v, out_hbm.at[pl.ds(base, BPW)])

    return sc_score(xt, v_tab, weight_char_emb, fc1_b)


# speedup vs baseline: 1.3678x; 1.1003x over previous
import functools

import jax
import jax.numpy as jnp
from jax import lax
from jax.experimental import pallas as pl
from jax.experimental.pallas import tpu as pltpu
from jax.experimental.pallas import tpu_sc as plsc

_LANES = 16
_NUM_CORES = 2
_NUM_SUBCORES = 16
_VOCAB_PAD = 1024
_NCHUNK = 4


def _vtable_tc_kernel(embT_ref, fcw_ref, out_ref):
    out_ref[...] = lax.dot_general(
        fcw_ref[...], embT_ref[...],
        (((1,), (0,)), ((), ())),
        preferred_element_type=jnp.float32,
    )


def kernel(input_x, char_emb, weight_char_emb, fc1_w, fc1_b):
    B, M = input_x.shape
    V, E = char_emb.shape
    NW = _NUM_CORES * _NUM_SUBCORES
    BPW = B // NW
    CW = BPW // _NCHUNK
    GPC = CW // _LANES
    MCH = (M + _LANES - 1) // _LANES

    v_tab = pl.pallas_call(
        _vtable_tc_kernel,
        out_shape=jax.ShapeDtypeStruct((1, V), jnp.float32),
    )(char_emb.T, fc1_w)
    v_tab = v_tab.reshape(V)

    xt = input_x.T

    mesh = plsc.VectorSubcoreMesh(core_axis_name="c", subcore_axis_name="s")

    @functools.partial(
        pl.kernel,
        out_type=jax.ShapeDtypeStruct((B,), jnp.float32),
        mesh=mesh,
        compiler_params=pltpu.CompilerParams(needs_layout_passes=False),
        scratch_types=[
            pltpu.VMEM((2, M, CW), jnp.int32),
            pltpu.VMEM((_VOCAB_PAD,), jnp.float32),
            pltpu.VMEM((MCH * _LANES,), jnp.float32),
            pltpu.VMEM((_LANES,), jnp.float32),
            pltpu.VMEM((BPW,), jnp.float32),
            pltpu.SemaphoreType.DMA,
            pltpu.SemaphoreType.DMA,
        ],
    )
    def sc_score(xt_hbm, v_hbm, w_hbm, b_hbm, out_hbm,
                 x_v, v_v, w_v, b_v, o_v, sem0, sem1):
        wid = lax.axis_index("s") * _NUM_CORES + lax.axis_index("c")
        base = wid * BPW
        sems = (sem0, sem1)

        pltpu.sync_copy(w_hbm, w_v.at[0:M])
        pltpu.sync_copy(b_hbm, b_v.at[0:1])
        pltpu.sync_copy(v_hbm, v_v.at[0:V])

        def start_chunk(c):
            return pltpu.async_copy(
                xt_hbm.at[:, pl.ds(base + c * CW, CW)],
                x_v.at[c % 2], sems[c % 2])

        pending = start_chunk(0)
        bias = b_v[pl.ds(0, _LANES)][0]
        MFULL = M // _LANES
        MTAIL = M % _LANES
        w_tail = w_v[pl.ds(MFULL * _LANES, _LANES)]

        for c in range(_NCHUNK):
            nxt = start_chunk(c + 1) if c + 1 < _NCHUNK else None
            pending.wait()
            xc = x_v.at[c % 2]

            def g_body(g, carry):
                gb = g * _LANES

                def mc_body(mc, acc):
                    wc = w_v[pl.ds(mc * _LANES, _LANES)]
                    mb = mc * _LANES
                    for i in range(_LANES):
                        idx = xc[mb + i, pl.ds(gb, _LANES)]
                        gv = plsc.load_gather(v_v, [idx])
                        acc = acc + gv * wc[i]
                    return acc

                acc = lax.fori_loop(
                    0, MFULL, mc_body, jnp.zeros((_LANES,), jnp.float32))
                for i in range(MTAIL):
                    idx = xc[MFULL * _LANES + i, pl.ds(gb, _LANES)]
                    gv = plsc.load_gather(v_v, [idx])
                    acc = acc + gv * w_tail[i]
                o_v[pl.ds(c * CW + gb, _LANES)] = acc + bias
                return carry

            lax.fori_loop(0, GPC, g_body, 0)
            pending = nxt

        pltpu.sync_copy(o_v, out_hbm.at[pl.ds(base, BPW)])

    return sc_score(xt, v_tab, weight_char_emb, fc1_b)


# speedup vs baseline: 1.4501x; 1.0602x over previous
import functools

import jax
import jax.numpy as jnp
from jax import lax
from jax.experimental import pallas as pl
from jax.experimental.pallas import tpu as pltpu
from jax.experimental.pallas import tpu_sc as plsc

_LANES = 16
_NUM_CORES = 2
_NUM_SUBCORES = 16
_VOCAB_PAD = 1024
_NCHUNK = 4


def _vtable_tc_kernel(embT_ref, fcw_ref, out_ref):
    out_ref[...] = lax.dot_general(
        fcw_ref[...], embT_ref[...],
        (((1,), (0,)), ((), ())),
        preferred_element_type=jnp.float32,
    )


def kernel(input_x, char_emb, weight_char_emb, fc1_w, fc1_b):
    B, M = input_x.shape
    V, E = char_emb.shape
    NW = _NUM_CORES * _NUM_SUBCORES
    BPW = B // NW
    CW = BPW // _NCHUNK
    GPC = CW // _LANES
    MCH = (M + _LANES - 1) // _LANES

    v_tab = pl.pallas_call(
        _vtable_tc_kernel,
        out_shape=jax.ShapeDtypeStruct((1, V), jnp.float32),
    )(char_emb.T, fc1_w)
    v_tab = v_tab.reshape(V)

    xt = input_x.T

    mesh = plsc.VectorSubcoreMesh(core_axis_name="c", subcore_axis_name="s")

    @functools.partial(
        pl.kernel,
        out_type=jax.ShapeDtypeStruct((B,), jnp.float32),
        mesh=mesh,
        compiler_params=pltpu.CompilerParams(needs_layout_passes=False),
        scratch_types=[
            pltpu.VMEM((2, M, CW), jnp.int32),
            pltpu.VMEM((_VOCAB_PAD,), jnp.float32),
            pltpu.VMEM((_VOCAB_PAD * _LANES,), jnp.float32),
            pltpu.VMEM((MCH * _LANES,), jnp.float32),
            pltpu.VMEM((_LANES,), jnp.float32),
            pltpu.VMEM((BPW,), jnp.float32),
            pltpu.SemaphoreType.DMA,
            pltpu.SemaphoreType.DMA,
        ],
    )
    def sc_score(xt_hbm, v_hbm, w_hbm, b_hbm, out_hbm,
                 x_v, v_v, vrep_v, w_v, b_v, o_v, sem0, sem1):
        wid = lax.axis_index("s") * _NUM_CORES + lax.axis_index("c")
        base = wid * BPW
        sems = (sem0, sem1)

        def start_chunk(c):
            return pltpu.async_copy(
                xt_hbm.at[:, pl.ds(base + c * CW, CW)],
                x_v.at[c % 2], sems[c % 2])

        pending = start_chunk(0)
        pltpu.sync_copy(w_hbm, w_v.at[0:M])
        pltpu.sync_copy(b_hbm, b_v.at[0:1])
        pltpu.sync_copy(v_hbm, v_v.at[0:V])

        def rep_body(jc, carry):
            chunk = v_v[pl.ds(jc * _LANES, _LANES)]
            for i in range(_LANES):
                vrep_v[pl.ds((jc * _LANES + i) * _LANES, _LANES)] = (
                    jnp.full((_LANES,), chunk[i], jnp.float32))
            return carry

        lax.fori_loop(0, (V + _LANES - 1) // _LANES, rep_body, 0)
        lane_iota = lax.iota(jnp.int32, _LANES)
        bias = b_v[pl.ds(0, _LANES)][0]
        MFULL = M // _LANES
        MTAIL = M % _LANES
        w_tail = w_v[pl.ds(MFULL * _LANES, _LANES)]

        for c in range(_NCHUNK):
            nxt = start_chunk(c + 1) if c + 1 < _NCHUNK else None
            pending.wait()
            xc = x_v.at[c % 2]

            def g_body(g, carry):
                gb = g * _LANES

                def mc_body(mc, acc):
                    wc = w_v[pl.ds(mc * _LANES, _LANES)]
                    mb = mc * _LANES
                    for i in range(_LANES):
                        idx = xc[mb + i, pl.ds(gb, _LANES)]
                        gv = plsc.load_gather(
                            vrep_v, [(idx << 4) + lane_iota])
                        acc = acc + gv * wc[i]
                    return acc

                acc = lax.fori_loop(
                    0, MFULL, mc_body, jnp.zeros((_LANES,), jnp.float32))
                for i in range(MTAIL):
                    idx = xc[MFULL * _LANES + i, pl.ds(gb, _LANES)]
                    gv = plsc.load_gather(
                        vrep_v, [(idx << 4) + lane_iota])
                    acc = acc + gv * w_tail[i]
                o_v[pl.ds(c * CW + gb, _LANES)] = acc + bias
                return carry

            lax.fori_loop(0, GPC, g_body, 0)
            pending = nxt

        pltpu.sync_copy(o_v, out_hbm.at[pl.ds(base, BPW)])

    return sc_score(xt, v_tab, weight_char_emb, fc1_b)


# speedup vs baseline: 1.4790x; 1.0199x over previous
import functools

import jax
import jax.numpy as jnp
from jax import lax
from jax.experimental import pallas as pl
from jax.experimental.pallas import tpu as pltpu
from jax.experimental.pallas import tpu_sc as plsc

_LANES = 16
_NUM_CORES = 2
_NUM_SUBCORES = 16
_VOCAB_PAD = 1024
_NCHUNK = 4


def _vtable_tc_kernel(embT_ref, fcw_ref, out_ref):
    out_ref[...] = lax.dot_general(
        fcw_ref[...], embT_ref[...],
        (((1,), (0,)), ((), ())),
        preferred_element_type=jnp.float32,
    )


def kernel(input_x, char_emb, weight_char_emb, fc1_w, fc1_b):
    B, M = input_x.shape
    V, E = char_emb.shape
    NW = _NUM_CORES * _NUM_SUBCORES
    BPW = B // NW
    CW = BPW // _NCHUNK
    GPC = CW // _LANES
    MCH = (M + _LANES - 1) // _LANES

    v_tab = pl.pallas_call(
        _vtable_tc_kernel,
        out_shape=jax.ShapeDtypeStruct((1, V), jnp.float32),
    )(char_emb.T, fc1_w)
    v_tab = v_tab.reshape(V)

    xt = input_x.T

    mesh = plsc.VectorSubcoreMesh(core_axis_name="c", subcore_axis_name="s")

    @functools.partial(
        pl.kernel,
        out_type=jax.ShapeDtypeStruct((B,), jnp.float32),
        mesh=mesh,
        compiler_params=pltpu.CompilerParams(needs_layout_passes=False),
        scratch_types=[
            pltpu.VMEM((2, M, CW), jnp.int32),
            pltpu.VMEM((_VOCAB_PAD,), jnp.float32),
            pltpu.VMEM((_VOCAB_PAD * _LANES,), jnp.float32),
            pltpu.VMEM((MCH * _LANES,), jnp.float32),
            pltpu.VMEM((_LANES,), jnp.float32),
            pltpu.VMEM((BPW,), jnp.float32),
            pltpu.SemaphoreType.DMA,
            pltpu.SemaphoreType.DMA,
        ],
    )
    def sc_score(xt_hbm, v_hbm, w_hbm, b_hbm, out_hbm,
                 x_v, v_v, vrep_v, w_v, b_v, o_v, sem0, sem1):
        wid = lax.axis_index("s") * _NUM_CORES + lax.axis_index("c")
        base = wid * BPW

        def start_chunk(c):
            return pltpu.async_copy(
                xt_hbm.at[:, pl.ds(base + c * CW, CW)],
                x_v.at[c & 1], sem0)

        start_chunk(0)
        pltpu.sync_copy(w_hbm, w_v.at[0:M])
        pltpu.sync_copy(b_hbm, b_v.at[0:1])
        pltpu.sync_copy(v_hbm, v_v.at[0:V])

        def rep_body(jc, carry):
            chunk = v_v[pl.ds(jc * _LANES, _LANES)]
            for i in range(_LANES):
                vrep_v[pl.ds((jc * _LANES + i) * _LANES, _LANES)] = (
                    jnp.full((_LANES,), chunk[i], jnp.float32))
            return carry

        lax.fori_loop(0, (V + _LANES - 1) // _LANES, rep_body, 0)
        lane_iota = lax.iota(jnp.int32, _LANES)
        bias = b_v[pl.ds(0, _LANES)][0]
        MFULL = M // _LANES
        MTAIL = M % _LANES
        w_tail = w_v[pl.ds(MFULL * _LANES, _LANES)]

        def g_body(g, carry):
            c = g // GPC
            gc = g % GPC

            @pl.when(jnp.logical_and(gc == 0, c + 1 < _NCHUNK))
            def _():
                start_chunk(c + 1)

            @pl.when(gc == 0)
            def _():
                pltpu.make_async_copy(
                    xt_hbm.at[:, pl.ds(base, CW)], x_v.at[0], sem0).wait()

            xc = x_v.at[c & 1]
            gb = gc * _LANES

            def mc_body(mc, acc):
                wc = w_v[pl.ds(mc * _LANES, _LANES)]
                mb = mc * _LANES
                for i in range(_LANES):
                    idx = xc[mb + i, pl.ds(gb, _LANES)]
                    gv = plsc.load_gather(
                        vrep_v, [(idx << 4) + lane_iota])
                    acc = acc + gv * wc[i]
                return acc

            acc = lax.fori_loop(
                0, MFULL, mc_body, jnp.zeros((_LANES,), jnp.float32))
            for i in range(MTAIL):
                idx = xc[MFULL * _LANES + i, pl.ds(gb, _LANES)]
                gv = plsc.load_gather(
                    vrep_v, [(idx << 4) + lane_iota])
                acc = acc + gv * w_tail[i]
            o_v[pl.ds(g * _LANES, _LANES)] = acc + bias
            return carry

        lax.fori_loop(0, _NCHUNK * GPC, g_body, 0)

        pltpu.sync_copy(o_v, out_hbm.at[pl.ds(base, BPW)])

    return sc_score(xt, v_tab, weight_char_emb, fc1_b)
